# baseline (device time: 149407 ns/iter reference)
import jax
import jax.numpy as jnp
from jax import lax
from jax.experimental import pallas as pl
from jax.experimental.pallas import tpu as pltpu

N_Z = 4


def kernel(x):
    m_per, n = x.shape
    half = m_per // 2

    def body(x_ref, out_ref, z_send, z_recv, c_send, c_recv):
        my_x = lax.axis_index("x")
        my_y = lax.axis_index("y")
        my_z = lax.axis_index("z")

        barrier = pltpu.get_barrier_semaphore()
        for d in range(1, N_Z):
            for xx in (my_x, 1 - my_x):
                pl.semaphore_signal(
                    barrier, inc=1,
                    device_id=(xx, my_y, (my_z + d) % N_Z),
                    device_id_type=pl.DeviceIdType.MESH,
                )
        pl.semaphore_wait(barrier, 2 * (N_Z - 1))

        def rdma(src, dst, ssem, rsem, dev):
            return pltpu.make_async_remote_copy(
                src_ref=src, dst_ref=dst, send_sem=ssem, recv_sem=rsem,
                device_id=dev, device_id_type=pl.DeviceIdType.MESH,
            )

        def out_half(z_origin, xh):
            return out_ref.at[pl.ds(z_origin * m_per + xh * half, half), :]

        my_half = x_ref.at[pl.ds(my_x * half, half), :]

        for d in range(1, N_Z):
            rdma(
                my_half, out_half(my_z, my_x),
                z_send.at[d - 1], z_recv.at[d - 1],
                (my_x, my_y, (my_z + d) % N_Z),
            ).start()
            rdma(
                my_half, out_half(my_z, my_x),
                c_send.at[d - 1], c_recv.at[d - 1],
                (1 - my_x, my_y, (my_z + d) % N_Z),
            ).start()

        out_ref[pl.ds(my_z * m_per, m_per), :] = x_ref[:, :]

        for d in range(1, N_Z):
            org = (my_z - d) % N_Z
            rdma(
                out_half(org, my_x), out_half(org, my_x),
                z_send.at[d - 1], z_recv.at[d - 1],
                (my_x, my_y, org),
            ).wait_recv()
            rdma(
                out_half(org, 1 - my_x), out_half(org, 1 - my_x),
                c_send.at[d - 1], c_recv.at[d - 1],
                (1 - my_x, my_y, org),
            ).wait_recv()

        for d in range(1, N_Z):
            rdma(
                my_half, out_half(my_z, my_x),
                z_send.at[d - 1], z_recv.at[d - 1],
                (my_x, my_y, (my_z + d) % N_Z),
            ).wait_send()
            rdma(
                my_half, out_half(my_z, my_x),
                c_send.at[d - 1], c_recv.at[d - 1],
                (1 - my_x, my_y, (my_z + d) % N_Z),
            ).wait_send()

    return pl.pallas_call(
        body,
        out_shape=jax.ShapeDtypeStruct((N_Z * m_per, n), x.dtype),
        in_specs=[pl.BlockSpec(memory_space=pltpu.VMEM)],
        out_specs=pl.BlockSpec(memory_space=pltpu.VMEM),
        scratch_shapes=[
            pltpu.SemaphoreType.DMA((N_Z - 1,)),
            pltpu.SemaphoreType.DMA((N_Z - 1,)),
            pltpu.SemaphoreType.DMA((N_Z - 1,)),
            pltpu.SemaphoreType.DMA((N_Z - 1,)),
        ],
        compiler_params=pltpu.CompilerParams(collective_id=0),
    )(x)


# device time: 128046 ns/iter; 1.1668x vs baseline; 1.1668x over previous
import jax
import jax.numpy as jnp
from jax import lax
from jax.experimental import pallas as pl
from jax.experimental.pallas import tpu as pltpu

N_Z = 4


def kernel(x):
    m_per, n = x.shape
    half = m_per // 2

    def body(x_ref, out_ref, z_send, z_recv, xf_send, xf_recv):
        my_x = lax.axis_index("x")
        my_y = lax.axis_index("y")
        my_z = lax.axis_index("z")

        barrier = pltpu.get_barrier_semaphore()
        for d in range(1, N_Z):
            pl.semaphore_signal(
                barrier, inc=1,
                device_id=(my_x, my_y, (my_z + d) % N_Z),
                device_id_type=pl.DeviceIdType.MESH,
            )
        pl.semaphore_signal(
            barrier, inc=1, device_id=(1 - my_x, my_y, my_z),
            device_id_type=pl.DeviceIdType.MESH,
        )
        pl.semaphore_wait(barrier, N_Z)

        def rdma(src, dst, ssem, rsem, dev):
            return pltpu.make_async_remote_copy(
                src_ref=src, dst_ref=dst, send_sem=ssem, recv_sem=rsem,
                device_id=dev, device_id_type=pl.DeviceIdType.MESH,
            )

        def variant(k, xx):
            xnbr = (1 - xx, my_y, k)
            own = x_ref.at[xx * half:(xx + 1) * half, :]

            def out_half(org, xh):
                r0 = org * m_per + xh * half
                return out_ref.at[r0:r0 + half, :]

            for d in range(1, N_Z):
                rdma(
                    own, out_half(k, xx),
                    z_send.at[d - 1], z_recv.at[d - 1],
                    (xx, my_y, (k + d) % N_Z),
                ).start()

            out_ref[k * m_per:(k + 1) * m_per, :] = x_ref[:, :]

            for d in range(1, N_Z):
                org = (k - d) % N_Z
                rdma(
                    out_half(org, xx), out_half(org, xx),
                    z_send.at[d - 1], z_recv.at[d - 1], xnbr,
                ).wait_recv()
                rdma(
                    out_half(org, xx), out_half(org, xx),
                    xf_send.at[d - 1], xf_recv.at[d - 1], xnbr,
                ).start()

            for d in range(1, N_Z):
                org = (k - d) % N_Z
                rdma(
                    out_half(org, 1 - xx), out_half(org, 1 - xx),
                    xf_send.at[d - 1], xf_recv.at[d - 1], xnbr,
                ).wait_recv()

            for d in range(1, N_Z):
                org = (k - d) % N_Z
                rdma(
                    own, out_half(k, xx),
                    z_send.at[d - 1], z_recv.at[d - 1],
                    (xx, my_y, (k + d) % N_Z),
                ).wait_send()
                rdma(
                    out_half(org, xx), out_half(org, xx),
                    xf_send.at[d - 1], xf_recv.at[d - 1], xnbr,
                ).wait_send()

        for k in range(N_Z):
            for xx in range(2):
                @pl.when((my_z == k) & (my_x == xx))
                def _(k=k, xx=xx):
                    variant(k, xx)

    return pl.pallas_call(
        body,
        out_shape=jax.ShapeDtypeStruct((N_Z * m_per, n), x.dtype),
        in_specs=[pl.BlockSpec(memory_space=pltpu.VMEM)],
        out_specs=pl.BlockSpec(memory_space=pltpu.VMEM),
        scratch_shapes=[
            pltpu.SemaphoreType.DMA((N_Z - 1,)),
            pltpu.SemaphoreType.DMA((N_Z - 1,)),
            pltpu.SemaphoreType.DMA((N_Z - 1,)),
            pltpu.SemaphoreType.DMA((N_Z - 1,)),
        ],
        compiler_params=pltpu.CompilerParams(collective_id=0),
    )(x)


# device time: 108434 ns/iter; 1.3779x vs baseline; 1.1809x over previous
import jax
import jax.numpy as jnp
from jax import lax
from jax.experimental import pallas as pl
from jax.experimental.pallas import tpu as pltpu

N_Z = 4
N_S = N_Z - 1


def kernel(x):
    m_per, n = x.shape
    half = m_per // 2

    def body(
        x_ref,
        out_ref,
        zr_send,
        zr_recv,
        zl_send,
        zl_recv,
        xr_send,
        xr_recv,
        xl_send,
        xl_recv,
    ):
        my_x = lax.axis_index("x")
        my_y = lax.axis_index("y")
        my_z = lax.axis_index("z")
        xn = (1 - my_x, my_y, my_z)
        zr = (my_x, my_y, (my_z + 1) % N_Z)
        zl = (my_x, my_y, (my_z - 1) % N_Z)

        barrier = pltpu.get_barrier_semaphore()
        for nbr in (xn, zr, zl):
            pl.semaphore_signal(
                barrier, inc=1, device_id=nbr,
                device_id_type=pl.DeviceIdType.MESH,
            )
        pl.semaphore_wait(barrier, 3)

        def clamp(v):
            return jnp.clip(v, 0, N_Z - 1)

        def half_ref(ref, z_origin, xh):
            return ref.at[pl.ds(z_origin * m_per + xh * half, half), :]

        def rdma(src, dst, ssem, rsem, dev):
            return pltpu.make_async_remote_copy(
                src_ref=src, dst_ref=dst, send_sem=ssem, recv_sem=rsem,
                device_id=dev, device_id_type=pl.DeviceIdType.MESH,
            )

        def zr_send_el(s):
            return (my_z < N_Z - 1) & (s <= my_z)

        def zr_recv_el(s):
            return (my_z >= 1) & (s <= my_z - 1)

        def zl_send_el(s):
            return (my_z >= 1) & (s <= N_Z - 1 - my_z)

        def zl_recv_el(s):
            return (my_z <= N_Z - 2) & (s <= N_Z - 2 - my_z)

        def zr_send_org(s):
            return clamp(my_z - s)

        def zr_recv_org(s):
            return clamp(my_z - 1 - s)

        def zl_send_org(s):
            return clamp(my_z + s)

        def zl_recv_org(s):
            return clamp(my_z + 1 + s)

        def zr_rdma(s):
            org = zr_send_org(s)
            return rdma(half_ref(out_ref, org, my_x),
                        half_ref(out_ref, org, my_x),
                        zr_send.at[s], zr_recv.at[s], zr)

        def zl_rdma(s):
            org = zl_send_org(s)
            return rdma(half_ref(out_ref, org, my_x),
                        half_ref(out_ref, org, my_x),
                        zl_send.at[s], zl_recv.at[s], zl)

        def zr_recv_rdma(s):
            org = zr_recv_org(s)
            return rdma(half_ref(out_ref, org, my_x),
                        half_ref(out_ref, org, my_x),
                        zr_send.at[s], zr_recv.at[s], zl)

        def zl_recv_rdma(s):
            org = zl_recv_org(s)
            return rdma(half_ref(out_ref, org, my_x),
                        half_ref(out_ref, org, my_x),
                        zl_send.at[s], zl_recv.at[s], zr)

        def xr_rdma(s):
            org = zr_recv_org(s)
            return rdma(half_ref(out_ref, org, my_x),
                        half_ref(out_ref, org, my_x),
                        xr_send.at[s], xr_recv.at[s], xn)

        def xl_rdma(s):
            org = zl_recv_org(s)
            return rdma(half_ref(out_ref, org, my_x),
                        half_ref(out_ref, org, my_x),
                        xl_send.at[s], xl_recv.at[s], xn)

        def xr_in_rdma(s):
            org = zr_recv_org(s)
            return rdma(half_ref(out_ref, org, 1 - my_x),
                        half_ref(out_ref, org, 1 - my_x),
                        xr_send.at[s], xr_recv.at[s], xn)

        def xl_in_rdma(s):
            org = zl_recv_org(s)
            return rdma(half_ref(out_ref, org, 1 - my_x),
                        half_ref(out_ref, org, 1 - my_x),
                        xl_send.at[s], xl_recv.at[s], xn)

        out_ref[pl.ds(my_z * m_per, m_per), :] = x_ref[:, :]

        @pl.when(zr_send_el(0))
        def _():
            rdma(x_ref.at[pl.ds(my_x * half, half), :],
                 half_ref(out_ref, my_z, my_x),
                 zr_send.at[0], zr_recv.at[0], zr).start()

        @pl.when(zl_send_el(0))
        def _():
            rdma(x_ref.at[pl.ds(my_x * half, half), :],
                 half_ref(out_ref, my_z, my_x),
                 zl_send.at[0], zl_recv.at[0], zl).start()

        for s in range(1, N_S):
            @pl.when(zr_recv_el(s - 1))
            def _(s=s):
                zr_recv_rdma(s - 1).wait_recv()

            @pl.when(zr_send_el(s))
            def _(s=s):
                zr_rdma(s).start()

            @pl.when(zr_recv_el(s - 1))
            def _(s=s):
                xr_rdma(s - 1).start()

            @pl.when(zl_recv_el(s - 1))
            def _(s=s):
                zl_recv_rdma(s - 1).wait_recv()

            @pl.when(zl_send_el(s))
            def _(s=s):
                zl_rdma(s).start()

            @pl.when(zl_recv_el(s - 1))
            def _(s=s):
                xl_rdma(s - 1).start()

        @pl.when(zr_recv_el(N_S - 1))
        def _():
            zr_recv_rdma(N_S - 1).wait_recv()
            xr_rdma(N_S - 1).start()

        @pl.when(zl_recv_el(N_S - 1))
        def _():
            zl_recv_rdma(N_S - 1).wait_recv()
            xl_rdma(N_S - 1).start()

        for s in range(N_S):
            @pl.when(zr_recv_el(s))
            def _(s=s):
                xr_in_rdma(s).wait_recv()

            @pl.when(zl_recv_el(s))
            def _(s=s):
                xl_in_rdma(s).wait_recv()

        for s in range(N_S):
            @pl.when(zr_send_el(s))
            def _(s=s):
                zr_rdma(s).wait_send()

            @pl.when(zl_send_el(s))
            def _(s=s):
                zl_rdma(s).wait_send()

            @pl.when(zr_recv_el(s))
            def _(s=s):
                xr_rdma(s).wait_send()

            @pl.when(zl_recv_el(s))
            def _(s=s):
                xl_rdma(s).wait_send()

    return pl.pallas_call(
        body,
        out_shape=jax.ShapeDtypeStruct((N_Z * m_per, n), x.dtype),
        in_specs=[pl.BlockSpec(memory_space=pltpu.VMEM)],
        out_specs=pl.BlockSpec(memory_space=pltpu.VMEM),
        scratch_shapes=[
            pltpu.SemaphoreType.DMA((N_S,)),
            pltpu.SemaphoreType.DMA((N_S,)),
            pltpu.SemaphoreType.DMA((N_S,)),
            pltpu.SemaphoreType.DMA((N_S,)),
            pltpu.SemaphoreType.DMA((N_S,)),
            pltpu.SemaphoreType.DMA((N_S,)),
            pltpu.SemaphoreType.DMA((N_S,)),
            pltpu.SemaphoreType.DMA((N_S,)),
        ],
        compiler_params=pltpu.CompilerParams(collective_id=0),
    )(x)
